# baseline (speedup 1.0000x reference)
import functools

import jax
import jax.numpy as jnp
from jax import lax
from jax.experimental import pallas as pl
from jax.experimental.pallas import tpu as pltpu
from jax.experimental.pallas import tpu_sc as plsc

EMB = 300
D = 384
DC = 96
NCH = 4
NP = 10240
NE = 161792
NEPT = NE // 16
NB = NEPT // 128
RPT = NP // 16
NMP = 512
NEM = 2048
NG = 64
DUMP = 10200

_mesh = plsc.VectorSubcoreMesh(core_axis_name="c", subcore_axis_name="s",
                               num_cores=2, num_subcores=16)
_sc_params = pltpu.CompilerParams(needs_layout_passes=False,
                                  use_tc_tiling_on_sc=False)


def _fill(ref, value):
    for k in range(8):
        ref[pl.ds(k * 16, 16)] = jnp.full((16,), value, jnp.float32)


def _rsqrt16(x):
    i = plsc.bitcast(x, jnp.int32)
    i = jnp.int32(0x5F3759DF) - (i >> 1)
    y = plsc.bitcast(i, jnp.float32)
    for _ in range(3):
        y = y * (1.5 - 0.5 * x * y * y)
    return y


@functools.partial(
    pl.kernel,
    out_type=(jax.ShapeDtypeStruct((NE,), jnp.float32),
              jax.ShapeDtypeStruct((NP,), jnp.float32)),
    mesh=_mesh,
    scratch_types=[
        pltpu.VMEM((128,), jnp.int32),
        pltpu.VMEM((128,), jnp.int32),
        pltpu.VMEM((128,), jnp.float32),
        pltpu.VMEM((128,), jnp.float32),
        pltpu.VMEM((NP,), jnp.float32),
        pltpu.VMEM_SHARED((NP,), jnp.float32),
        pltpu.SemaphoreType.DMA,
    ],
    compiler_params=_sc_params,
)
def _sc_deg_norm(src_hbm, dst_hbm, norm_hbm, invd_hbm,
                 si_v, di_v, ones_v, st_v, degl_v, deg_sh, sem):
    c = lax.axis_index("c")
    s = lax.axis_index("s")

    @pl.when(c == 0)
    def _():
        _fill(ones_v, 1.0)
        _fill(st_v, 0.0)
        for i in range(RPT // 128):
            pltpu.sync_copy(st_v, deg_sh.at[pl.ds(s * RPT + i * 128, 128)])
        plsc.subcore_barrier()

        def blk(b, carry):
            off = s * NEPT + b * 128
            pltpu.sync_copy(dst_hbm.at[pl.ds(off, 128)], di_v)
            pltpu.sync_copy(ones_v, deg_sh.at[di_v], add=True)
            return carry

        lax.fori_loop(0, NB, blk, 0)
        plsc.subcore_barrier()
        pltpu.sync_copy(deg_sh, degl_v)

        for i in range(RPT // 128):
            base = s * RPT + i * 128
            for k in range(8):
                dv = degl_v[pl.ds(base + k * 16, 16)] + 1.0
                st_v[pl.ds(k * 16, 16)] = 1.0 / dv
            pltpu.sync_copy(st_v, invd_hbm.at[pl.ds(base, 128)])

        def nblk(b, carry):
            off = s * NEPT + b * 128
            pltpu.sync_copy(src_hbm.at[pl.ds(off, 128)], si_v)
            pltpu.sync_copy(dst_hbm.at[pl.ds(off, 128)], di_v)
            for k in range(8):
                sidx = si_v[pl.ds(k * 16, 16)]
                didx = di_v[pl.ds(k * 16, 16)]
                dsv = plsc.load_gather(degl_v, [sidx]) + 1.0
                ddv = plsc.load_gather(degl_v, [didx]) + 1.0
                st_v[pl.ds(k * 16, 16)] = _rsqrt16(dsv * ddv)
            pltpu.sync_copy(st_v, norm_hbm.at[pl.ds(off, 128)])
            return carry

        lax.fori_loop(0, NB, nblk, 0)


def _make_sc_edge(relu, layer):
    @functools.partial(
        pl.kernel,
        out_type=jax.ShapeDtypeStruct((NCH * NP, DC), jnp.float32),
        mesh=_mesh,
        scratch_types=[
            pltpu.VMEM((128,), jnp.int32),
            pltpu.VMEM((128,), jnp.int32),
            pltpu.VMEM((128,), jnp.float32),
            pltpu.VMEM((128, DC), jnp.float32),
            pltpu.VMEM((128, DC), jnp.float32),
            pltpu.VMEM((128, DC), jnp.float32),
            pltpu.VMEM_SHARED((NP, DC), jnp.float32),
            pltpu.SemaphoreType.DMA,
        ],
        name=f"sc_edge_l{layer}",
        compiler_params=_sc_params,
    )
    def k(hw_hbm, e_hbm, src_hbm, dst_hbm, norm_hbm, invd_hbm, h_hbm, out_hbm,
          si_v, di_v, nv_v, rows_v, ev_v, msg_v, agg_sh, sem):
        c = lax.axis_index("c")
        s = lax.axis_index("s")

        for p in range(2):
            ch = 2 * p + c
            rowbase = ch * NP
            ebase = (NCH * layer + ch) * NE

            def zrow(j, carry):
                for k2 in range(DC // 16):
                    msg_v[j, pl.ds(k2 * 16, 16)] = jnp.zeros((16,), jnp.float32)
                return carry

            lax.fori_loop(0, 128, zrow, 0)
            for i in range(RPT // 128):
                pltpu.sync_copy(msg_v, agg_sh.at[pl.ds(s * RPT + i * 128, 128)])
            plsc.subcore_barrier()

            def blk(b, carry):
                off = s * NEPT + b * 128
                pltpu.sync_copy(dst_hbm.at[pl.ds(off, 128)], di_v)
                pltpu.sync_copy(norm_hbm.at[pl.ds(off, 128)], nv_v)
                pltpu.sync_copy(e_hbm.at[pl.ds(ebase + off, 128)], ev_v)
                pltpu.sync_copy(src_hbm.at[pl.ds(off, 128)], si_v)
                for k2 in range(8):
                    si_v[pl.ds(k2 * 16, 16)] = si_v[pl.ds(k2 * 16, 16)] + rowbase
                pltpu.async_copy(hw_hbm.at[si_v], rows_v, sem).wait()

                def row(j, carry2):
                    jj = jnp.full((16,), 0, jnp.int32) + j
                    nrm = plsc.load_gather(nv_v, [jj])
                    for k2 in range(DC // 16):
                        v = rows_v[j, pl.ds(k2 * 16, 16)] + ev_v[j, pl.ds(k2 * 16, 16)]
                        msg_v[j, pl.ds(k2 * 16, 16)] = jnp.maximum(v, 0.0) * nrm
                    return carry2

                lax.fori_loop(0, 128, row, 0)
                pltpu.sync_copy(msg_v, agg_sh.at[di_v], add=True)
                return carry

            lax.fori_loop(0, NB, blk, 0)
            plsc.subcore_barrier()

            for i in range(RPT // 128):
                r0 = s * RPT + i * 128
                pltpu.sync_copy(agg_sh.at[pl.ds(r0, 128)], rows_v)
                pltpu.sync_copy(hw_hbm.at[pl.ds(rowbase + r0, 128)], ev_v)
                pltpu.sync_copy(h_hbm.at[pl.ds(rowbase + r0, 128)], msg_v)
                pltpu.sync_copy(invd_hbm.at[pl.ds(r0, 128)], nv_v)

                def crow(j, carry):
                    jj = jnp.full((16,), 0, jnp.int32) + j
                    idv = plsc.load_gather(nv_v, [jj])
                    for k2 in range(DC // 16):
                        o = rows_v[j, pl.ds(k2 * 16, 16)] + ev_v[j, pl.ds(k2 * 16, 16)] * idv
                        if relu:
                            o = jnp.maximum(o, 0.0)
                        rows_v[j, pl.ds(k2 * 16, 16)] = o + msg_v[j, pl.ds(k2 * 16, 16)]
                    return carry

                lax.fori_loop(0, 128, crow, 0)
                pltpu.sync_copy(rows_v, out_hbm.at[pl.ds(rowbase + r0, 128)])

    return k


_sc_edge = [_make_sc_edge(l < 4, l) for l in range(5)]


def _mm_h_body(a_ref, w_ref, b_ref, o_ref):
    a = a_ref[...]
    w = w_ref[...]
    acc = jnp.dot(a[0], w[0, 0], preferred_element_type=jnp.float32)
    for k in range(1, NCH):
        acc = acc + jnp.dot(a[k], w[k, 0], preferred_element_type=jnp.float32)
    o_ref[...] = (acc + b_ref[0, 0:1, :])[None]


def _tc_matmul_h(h3, wr, bb):
    bm = 1024
    nb = NP // bm
    return pl.pallas_call(
        _mm_h_body,
        grid=(nb, NCH),
        in_specs=[
            pl.BlockSpec((NCH, bm, DC), lambda i, c: (0, i, 0)),
            pl.BlockSpec((NCH, 1, DC, DC), lambda i, c: (0, c, 0, 0)),
            pl.BlockSpec((1, 8, DC), lambda i, c: (c, 0, 0)),
        ],
        out_specs=pl.BlockSpec((1, bm, DC), lambda i, c: (c, i, 0)),
        out_shape=jax.ShapeDtypeStruct((NCH, NP, DC), jnp.float32),
    )(h3, wr, bb)


def _mm_e_body(a_ref, w_ref, o_ref):
    o_ref[...] = jnp.dot(a_ref[...], w_ref[0],
                         preferred_element_type=jnp.float32)[None]


def _tc_matmul_e(attr, wr):
    bm = 2048
    nb = NE // bm
    ng = 5 * NCH
    return pl.pallas_call(
        _mm_e_body,
        grid=(nb, ng),
        in_specs=[
            pl.BlockSpec((bm, 128), lambda i, g: (i, 0)),
            pl.BlockSpec((1, 128, DC), lambda i, g: (g, 0, 0)),
        ],
        out_specs=pl.BlockSpec((1, bm, DC), lambda i, g: (g, i, 0)),
        out_shape=jax.ShapeDtypeStruct((ng, NE, DC), jnp.float32),
    )(attr, wr)


def _segmax_atoms_body(x_ref, m_ref, o_ref):
    mb = pl.program_id(0)
    x = x_ref[0]
    ids = m_ref[:, 0:1]
    for jj in range(8):
        mask = ids == (mb * 8 + jj)
        red = jnp.max(jnp.where(mask, x, -3e38), axis=0, keepdims=True)
        red = jnp.where(red < -1e30, 0.0, red)
        o_ref[0, jj:jj + 1, :] = red


def _tc_segmax_atoms(h3, a2mb):
    return pl.pallas_call(
        _segmax_atoms_body,
        grid=(NMP // 8, NCH),
        in_specs=[
            pl.BlockSpec((1, NP, DC), lambda mb, c: (c, 0, 0)),
            pl.BlockSpec((NP, 128), lambda mb, c: (0, 0)),
        ],
        out_specs=pl.BlockSpec((1, 8, DC), lambda mb, c: (c, mb, 0)),
        out_shape=jax.ShapeDtypeStruct((NCH, NMP, DC), jnp.float32),
    )(h3, a2mb)


def _motif_body(hm_ref, s_ref, d_ref, a_ref, wg_ref, weg_ref, bg_ref, g_ref,
                wp_ref, bp_ref, o_ref):
    f32 = jnp.float32
    iota = lax.broadcasted_iota(jnp.int32, (NEM, NMP), 1)
    oh_s = (jnp.broadcast_to(s_ref[:, 0:1], (NEM, NMP)) == iota).astype(f32)
    oh_d = (jnp.broadcast_to(d_ref[:, 0:1], (NEM, NMP)) == iota).astype(f32)
    ones = jnp.ones((NEM, 128), f32)
    dn = (((0,), (0,)), ((), ()))
    degc = lax.dot_general(oh_d, ones, dn, preferred_element_type=f32) + 1.0
    deg_s = jnp.dot(oh_s, degc, preferred_element_type=f32)[:, 0:1]
    deg_d = jnp.dot(oh_d, degc, preferred_element_type=f32)[:, 0:1]
    nrm = lax.rsqrt(deg_s * deg_d)
    invd = 1.0 / degc[:, 0:1]
    hm = hm_ref[...]
    am = a_ref[...]
    for l in range(5):
        hw = jnp.dot(hm, wg_ref[l], preferred_element_type=f32) + bg_ref[l:l + 1, :]
        e = jnp.dot(am, weg_ref[l], preferred_element_type=f32)
        hw_s = jnp.dot(oh_s, hw, preferred_element_type=f32)
        msg = nrm * jnp.maximum(hw_s + e, 0.0)
        agg = lax.dot_general(oh_d, msg, dn, preferred_element_type=f32)
        out = agg + hw * invd
        if l < 4:
            out = jnp.maximum(out, 0.0)
        hm = out + hm
    gids = g_ref[:, 0:1]
    rows = []
    for g in range(NG):
        red = jnp.max(jnp.where(gids == g, hm, -3e38), axis=0, keepdims=True)
        rows.append(jnp.where(red < -1e30, 0.0, red))
    hg = jnp.concatenate(rows, axis=0)
    o_ref[...] = jnp.dot(hg, wp_ref[...], preferred_element_type=f32) + bp_ref[0:1, :]


def _tc_motif(hm, srcb, dstb, attrm, wg5, weg5, bg8, m2gb, wp, bp8):
    return pl.pallas_call(
        _motif_body,
        out_shape=jax.ShapeDtypeStruct((NG, 128), jnp.float32),
    )(hm, srcb, dstb, attrm, wg5, weg5, bg8, m2gb, wp, bp8)


def kernel(logits, atom_mask, atom_edge_index, atom_edge_attr, atom2motif,
           motif_edge_index, motif_edge_attr, motif2graph,
           Wm, Wem, bm, Wg, Weg, bg, Wp, bp):
    f32 = jnp.float32
    i32 = jnp.int32
    B, N, _ = logits.shape
    n_atoms = B * N

    h0 = jnp.where(atom_mask.reshape(-1, 1), logits.reshape(n_atoms, EMB), 0.0)
    h0 = jnp.pad(h0, ((0, NP - n_atoms), (0, D - EMB)))
    h3 = h0.reshape(NP, NCH, DC).transpose(1, 0, 2)

    ne0 = atom_edge_index.shape[1]
    src = jnp.pad(atom_edge_index[0].astype(i32), (0, NE - ne0))
    dst = jnp.pad(atom_edge_index[1].astype(i32), (0, NE - ne0),
                  constant_values=DUMP)
    attr = jnp.pad(atom_edge_attr.astype(f32), ((0, NE - ne0), (0, 128 - 16)))

    wm_r = jnp.pad(Wm, ((0, 0), (0, D - EMB), (0, D - EMB)))
    wm_r = wm_r.reshape(5, NCH, DC, NCH, DC).transpose(0, 1, 3, 2, 4)
    bm_b = jnp.broadcast_to(
        jnp.pad(bm, ((0, 0), (0, D - EMB))).reshape(5, NCH, 1, DC),
        (5, NCH, 8, DC))
    wem_r = jnp.pad(Wem, ((0, 0), (0, 128 - 16), (0, D - EMB)))
    wem_r = wem_r.reshape(5, 128, NCH, DC).transpose(0, 2, 1, 3)
    wem_r = wem_r.reshape(5 * NCH, 128, DC)

    a2m = jnp.pad(atom2motif.astype(i32), (0, NP - n_atoms),
                  constant_values=NMP - 1)
    a2mb = jnp.broadcast_to(a2m[:, None], (NP, 128))

    nem0 = motif_edge_index.shape[1]
    src_m = jnp.pad(motif_edge_index[0].astype(i32), (0, NEM - nem0),
                    constant_values=500)
    dst_m = jnp.pad(motif_edge_index[1].astype(i32), (0, NEM - nem0),
                    constant_values=501)
    srcb = jnp.broadcast_to(src_m[:, None], (NEM, 128))
    dstb = jnp.broadcast_to(dst_m[:, None], (NEM, 128))
    attrm = jnp.pad(motif_edge_attr.astype(f32), ((0, NEM - nem0), (0, 128 - 16)))
    wg5 = jnp.pad(Wg, ((0, 0), (0, D - EMB), (0, D - EMB)))
    weg5 = jnp.pad(Weg, ((0, 0), (0, 128 - 16), (0, D - EMB)))
    bg8 = jnp.pad(bg, ((0, 3), (0, D - EMB)))
    m2g = jnp.pad(motif2graph.astype(i32), (0, NMP - 500),
                  constant_values=NG - 1)
    m2gb = jnp.broadcast_to(m2g[:, None], (NMP, 128))
    wp = jnp.pad(Wp, ((0, D - EMB), (0, 128 - Wp.shape[1])))
    bp8 = jnp.broadcast_to(jnp.pad(bp, (0, 128 - bp.shape[0]))[None, :], (8, 128))

    norm, invd = _sc_deg_norm(src, dst)
    e_all = _tc_matmul_e(attr, wem_r)
    e_flat = e_all.reshape(5 * NCH * NE, DC)
    h = h3.reshape(NCH * NP, DC)
    for l in range(5):
        hw3 = _tc_matmul_h(h.reshape(NCH, NP, DC), wm_r[l], bm_b[l])
        hw = hw3.reshape(NCH * NP, DC)
        h = _sc_edge[l](hw, e_flat, src, dst, norm, invd, h)

    hm3 = _tc_segmax_atoms(h.reshape(NCH, NP, DC), a2mb)
    hm = hm3.transpose(1, 0, 2).reshape(NMP, D)
    out = _tc_motif(hm, srcb, dstb, attrm, wg5, weg5, bg8, m2gb, wp, bp8)
    return out[:50, :12]


# speedup vs baseline: 1.3225x; 1.3225x over previous
import functools

import jax
import jax.numpy as jnp
from jax import lax
from jax.experimental import pallas as pl
from jax.experimental.pallas import tpu as pltpu
from jax.experimental.pallas import tpu_sc as plsc

EMB = 300
D = 320
DC = 80
NCH = 4
NP = 10240
NE = 163840
NBE = NE // 128
NBT = NBE // 16
NPAIR = NBT // 2
RPT = NP // 16
NBR = NP // 128
NMP = 512
NEM = 2048
NG = 64
DUMP = 10200

_mesh = plsc.VectorSubcoreMesh(core_axis_name="c", subcore_axis_name="s",
                               num_cores=2, num_subcores=16)
_sc_params = pltpu.CompilerParams(needs_layout_passes=False,
                                  use_tc_tiling_on_sc=False)


def _fill(ref, value):
    for k in range(8):
        ref[pl.ds(k * 16, 16)] = jnp.full((16,), value, jnp.float32)


def _rsqrt16(x):
    i = plsc.bitcast(x, jnp.int32)
    i = jnp.int32(0x5F3759DF) - (i >> 1)
    y = plsc.bitcast(i, jnp.float32)
    for _ in range(3):
        y = y * (1.5 - 0.5 * x * y * y)
    return y


@functools.partial(
    pl.kernel,
    out_type=(jax.ShapeDtypeStruct((NBE, 128), jnp.float32),
              jax.ShapeDtypeStruct((NBR, 128), jnp.float32)),
    mesh=_mesh,
    scratch_types=[
        pltpu.VMEM((128,), jnp.int32),
        pltpu.VMEM((128,), jnp.int32),
        pltpu.VMEM((128,), jnp.float32),
        pltpu.VMEM((128,), jnp.float32),
        pltpu.VMEM((NP,), jnp.float32),
        pltpu.VMEM_SHARED((NP,), jnp.float32),
        pltpu.SemaphoreType.DMA,
    ],
    compiler_params=_sc_params,
)
def _sc_deg_norm(src_hbm, dst_hbm, norm_hbm, invd_hbm,
                 si_v, di_v, ones_v, st_v, degl_v, deg_sh, sem):
    c = lax.axis_index("c")
    s = lax.axis_index("s")

    _fill(ones_v, 1.0)
    _fill(st_v, 0.0)
    for i in range(RPT // 128):
        pltpu.sync_copy(st_v, deg_sh.at[pl.ds(s * RPT + i * 128, 128)])
    plsc.subcore_barrier()

    def blk(b, carry):
        pltpu.sync_copy(dst_hbm.at[s * NBT + b], di_v)
        pltpu.sync_copy(ones_v, deg_sh.at[di_v], add=True)
        return carry

    lax.fori_loop(0, NBT, blk, 0)
    plsc.subcore_barrier()
    pltpu.sync_copy(deg_sh, degl_v)

    @pl.when(c == 0)
    def _():
        for i in range(RPT // 128):
            base = s * RPT + i * 128
            for k in range(8):
                dv = degl_v[pl.ds(base + k * 16, 16)] + 1.0
                st_v[pl.ds(k * 16, 16)] = 1.0 / dv
            pltpu.sync_copy(st_v, invd_hbm.at[s * (RPT // 128) + i])

    def nblk(b, carry):
        g = (c * 16 + s) * (NBT // 2) + b
        pltpu.sync_copy(src_hbm.at[g], si_v)
        pltpu.sync_copy(dst_hbm.at[g], di_v)
        for k in range(8):
            sidx = si_v[pl.ds(k * 16, 16)]
            didx = di_v[pl.ds(k * 16, 16)]
            dsv = plsc.load_gather(degl_v, [sidx]) + 1.0
            ddv = plsc.load_gather(degl_v, [didx]) + 1.0
            st_v[pl.ds(k * 16, 16)] = _rsqrt16(dsv * ddv)
        pltpu.sync_copy(st_v, norm_hbm.at[g])
        return carry

    lax.fori_loop(0, NBT // 2, nblk, 0)


def _make_sc_edge(relu, layer):
    @functools.partial(
        pl.kernel,
        out_type=jax.ShapeDtypeStruct((NCH * NP, DC), jnp.float32),
        mesh=_mesh,
        scratch_types=[
            pltpu.VMEM((128,), jnp.int32),
            pltpu.VMEM((128,), jnp.int32),
            pltpu.VMEM((8, 128), jnp.int32),
            pltpu.VMEM((128,), jnp.float32),
            pltpu.VMEM((128,), jnp.float32),
            pltpu.VMEM((128,), jnp.float32),
            pltpu.VMEM((128, DC), jnp.float32),
            pltpu.VMEM((128, DC), jnp.float32),
            pltpu.VMEM((128, DC), jnp.float32),
            pltpu.VMEM((128, DC), jnp.float32),
            pltpu.VMEM((128, DC), jnp.float32),
            pltpu.VMEM((128, DC), jnp.float32),
            pltpu.VMEM_SHARED((NP, DC), jnp.float32),
            pltpu.SemaphoreType.DMA,
            pltpu.SemaphoreType.DMA,
            pltpu.SemaphoreType.DMA,
            pltpu.SemaphoreType.DMA,
            pltpu.SemaphoreType.DMA,
            pltpu.SemaphoreType.DMA,
        ],
        name=f"sc_edge_l{layer}",
        compiler_params=_sc_params,
    )
    def k(hw_hbm, e_hbm, src_hbm, dst_hbm, norm_hbm, invd_hbm, h_hbm, out_hbm,
          siA, siB, di_v, nvA, nvB, iv_v, rA, rB, eA, eB, mA, mB, agg_sh,
          isA, isB, gsA, gsB, ssA, ssB):
        c = lax.axis_index("c")
        s = lax.axis_index("s")


        def compute_block(rows_v, ev_v, nv_v, msg_v):
            def row(j, carry2):
                jj = jnp.full((16,), 0, jnp.int32) + j
                nrm = plsc.load_gather(nv_v, [jj])
                for k2 in range(DC // 16):
                    v = rows_v[j, pl.ds(k2 * 16, 16)] + ev_v[j, pl.ds(k2 * 16, 16)]
                    msg_v[j, pl.ds(k2 * 16, 16)] = jnp.maximum(v, 0.0) * nrm
                return carry2
            lax.fori_loop(0, 128, row, 0)

        for p in range(2):
            ch = 2 * p + c
            rowbase = ch * NP
            ebase = (NCH * layer + ch) * NE

            def zrow(j, carry):
                for k2 in range(DC // 16):
                    mA[j, pl.ds(k2 * 16, 16)] = jnp.zeros((16,), jnp.float32)
                return carry

            lax.fori_loop(0, 128, zrow, 0)
            for i in range(RPT // 128):
                pltpu.sync_copy(mA, agg_sh.at[pl.ds(s * RPT + i * 128, 128)])
            plsc.subcore_barrier()

            def step(b, si_p, si_q, nv_p, nv_q, r_p, r_q, e_p, e_q, m_p,
                     is_p, is_q, gs_p, gs_q, ss_p, skip_sw):
                pltpu.make_async_copy(hw_hbm.at[si_p], r_p, gs_p).wait()
                pltpu.make_async_copy(e_hbm.at[pl.ds(ebase, 128)], e_p, gs_p).wait()
                pltpu.make_async_copy(norm_hbm.at[0], nv_p, gs_p).wait()
                @pl.when(b + 2 < NBT)
                def _():
                    pltpu.async_copy(src_hbm.at[s * NBT + b + 2], si_p, is_p)
                    pltpu.async_copy(dst_hbm.at[s * NBT + b + 2],
                                     di_v.at[lax.rem(b + 2, 8)], is_p)
                @pl.when(b + 1 < NBT)
                def _():
                    pltpu.make_async_copy(src_hbm.at[0], si_q, is_q).wait()
                    pltpu.make_async_copy(dst_hbm.at[0],
                                          di_v.at[0], is_q).wait()
                    def srow(k2, carry):
                        si_q[pl.ds(k2 * 16, 16)] = (
                            si_q[pl.ds(k2 * 16, 16)] + rowbase)
                        return carry
                    lax.fori_loop(0, 8, srow, 0)
                    off1 = s * NBT * 128 + (b + 1) * 128
                    pltpu.async_copy(hw_hbm.at[si_q], r_q, gs_q)
                    pltpu.async_copy(e_hbm.at[pl.ds(ebase + off1, 128)], e_q, gs_q)
                    pltpu.async_copy(norm_hbm.at[s * NBT + b + 1], nv_q, gs_q)
                @pl.when(jnp.logical_not(skip_sw))
                def _():
                    pltpu.make_async_copy(m_p, agg_sh.at[di_v.at[0]], ss_p).wait()
                compute_block(r_p, e_p, nv_p, m_p)
                pltpu.async_copy(m_p, agg_sh.at[di_v.at[lax.rem(b, 8)]],
                                 ss_p, add=True)

            pltpu.async_copy(src_hbm.at[s * NBT], siA, isA)
            pltpu.async_copy(dst_hbm.at[s * NBT], di_v.at[0], isA)
            pltpu.async_copy(src_hbm.at[s * NBT + 1], siB, isB)
            pltpu.async_copy(dst_hbm.at[s * NBT + 1], di_v.at[1], isB)
            pltpu.make_async_copy(src_hbm.at[0], siA, isA).wait()
            pltpu.make_async_copy(dst_hbm.at[0], di_v.at[0], isA).wait()

            def srow0(k2, carry):
                siA[pl.ds(k2 * 16, 16)] = siA[pl.ds(k2 * 16, 16)] + rowbase
                return carry
            lax.fori_loop(0, 8, srow0, 0)
            pltpu.async_copy(hw_hbm.at[siA], rA, gsA)
            pltpu.async_copy(e_hbm.at[pl.ds(ebase + s * NBT * 128, 128)], eA, gsA)
            pltpu.async_copy(norm_hbm.at[s * NBT], nvA, gsA)

            def pair(i, carry):
                step(2 * i, siA, siB, nvA, nvB, rA, rB, eA, eB, mA,
                     isA, isB, gsA, gsB, ssA, i == 0)
                step(2 * i + 1, siB, siA, nvB, nvA, rB, rA, eB, eA, mB,
                     isB, isA, gsB, gsA, ssB, i == 0)
                return carry

            lax.fori_loop(0, NPAIR, pair, 0)
            pltpu.make_async_copy(mA, agg_sh.at[di_v.at[0]], ssA).wait()
            pltpu.make_async_copy(mB, agg_sh.at[di_v.at[0]], ssB).wait()
            plsc.subcore_barrier()

            for i in range(RPT // 128):
                r0 = s * RPT + i * 128
                pltpu.sync_copy(agg_sh.at[pl.ds(r0, 128)], rA)
                pltpu.sync_copy(hw_hbm.at[pl.ds(rowbase + r0, 128)], eA)
                pltpu.sync_copy(h_hbm.at[pl.ds(rowbase + r0, 128)], mA)
                pltpu.sync_copy(invd_hbm.at[s * (RPT // 128) + i], iv_v)

                def crow(j, carry):
                    jj = jnp.full((16,), 0, jnp.int32) + j
                    idv = plsc.load_gather(iv_v, [jj])
                    for k2 in range(DC // 16):
                        o = rA[j, pl.ds(k2 * 16, 16)] + eA[j, pl.ds(k2 * 16, 16)] * idv
                        if relu:
                            o = jnp.maximum(o, 0.0)
                        rA[j, pl.ds(k2 * 16, 16)] = o + mA[j, pl.ds(k2 * 16, 16)]
                    return carry

                lax.fori_loop(0, 128, crow, 0)
                pltpu.sync_copy(rA, out_hbm.at[pl.ds(rowbase + r0, 128)])

    return k


_sc_edge = [_make_sc_edge(l < 4, l) for l in range(5)]


def _mm_h_body(a_ref, w_ref, b_ref, o_ref):
    a = a_ref[...]
    w = w_ref[...]
    acc = jnp.dot(a[0], w[0, 0], preferred_element_type=jnp.float32)
    for k in range(1, NCH):
        acc = acc + jnp.dot(a[k], w[k, 0], preferred_element_type=jnp.float32)
    o_ref[...] = (acc + b_ref[0, 0:1, :])[None]


def _tc_matmul_h(h3, wr, bb):
    bm = 1024
    nb = NP // bm
    return pl.pallas_call(
        _mm_h_body,
        grid=(nb, NCH),
        in_specs=[
            pl.BlockSpec((NCH, bm, DC), lambda i, c: (0, i, 0)),
            pl.BlockSpec((NCH, 1, DC, DC), lambda i, c: (0, c, 0, 0)),
            pl.BlockSpec((1, 8, DC), lambda i, c: (c, 0, 0)),
        ],
        out_specs=pl.BlockSpec((1, bm, DC), lambda i, c: (c, i, 0)),
        out_shape=jax.ShapeDtypeStruct((NCH, NP, DC), jnp.float32),
    )(h3, wr, bb)


def _mm_e_body(a_ref, w_ref, o_ref):
    o_ref[...] = jnp.dot(a_ref[...], w_ref[0],
                         preferred_element_type=jnp.float32)[None]


def _tc_matmul_e(attr, wr):
    bm = 2048
    nb = NE // bm
    ng = 5 * NCH
    return pl.pallas_call(
        _mm_e_body,
        grid=(nb, ng),
        in_specs=[
            pl.BlockSpec((bm, 128), lambda i, g: (i, 0)),
            pl.BlockSpec((1, 128, DC), lambda i, g: (g, 0, 0)),
        ],
        out_specs=pl.BlockSpec((1, bm, DC), lambda i, g: (g, i, 0)),
        out_shape=jax.ShapeDtypeStruct((ng, NE, DC), jnp.float32),
    )(attr, wr)


def _segmax_atoms_body(x_ref, m_ref, o_ref):
    mb = pl.program_id(0)
    x = x_ref[0]
    ids = m_ref[:, 0:1]
    for jj in range(8):
        mask = ids == (mb * 8 + jj)
        red = jnp.max(jnp.where(mask, x, -3e38), axis=0, keepdims=True)
        red = jnp.where(red < -1e30, 0.0, red)
        o_ref[0, jj:jj + 1, :] = red


def _tc_segmax_atoms(h3, a2mb):
    return pl.pallas_call(
        _segmax_atoms_body,
        grid=(NMP // 8, NCH),
        in_specs=[
            pl.BlockSpec((1, NP, DC), lambda mb, c: (c, 0, 0)),
            pl.BlockSpec((NP, 128), lambda mb, c: (0, 0)),
        ],
        out_specs=pl.BlockSpec((1, 8, DC), lambda mb, c: (c, mb, 0)),
        out_shape=jax.ShapeDtypeStruct((NCH, NMP, DC), jnp.float32),
    )(h3, a2mb)


def _motif_body(hm_ref, s_ref, d_ref, a_ref, wg_ref, weg_ref, bg_ref, g_ref,
                wp_ref, bp_ref, o_ref):
    f32 = jnp.float32
    iota = lax.broadcasted_iota(jnp.int32, (NEM, NMP), 1)
    oh_s = (jnp.broadcast_to(s_ref[:, 0:1], (NEM, NMP)) == iota).astype(f32)
    oh_d = (jnp.broadcast_to(d_ref[:, 0:1], (NEM, NMP)) == iota).astype(f32)
    ones = jnp.ones((NEM, 128), f32)
    dn = (((0,), (0,)), ((), ()))
    degc = lax.dot_general(oh_d, ones, dn, preferred_element_type=f32) + 1.0
    deg_s = jnp.dot(oh_s, degc, preferred_element_type=f32)[:, 0:1]
    deg_d = jnp.dot(oh_d, degc, preferred_element_type=f32)[:, 0:1]
    nrm = lax.rsqrt(deg_s * deg_d)
    invd = 1.0 / degc[:, 0:1]
    hm = hm_ref[...]
    am = a_ref[...]
    for l in range(5):
        hw = jnp.dot(hm, wg_ref[l], preferred_element_type=f32) + bg_ref[l:l + 1, :]
        e = jnp.dot(am, weg_ref[l], preferred_element_type=f32)
        hw_s = jnp.dot(oh_s, hw, preferred_element_type=f32)
        msg = nrm * jnp.maximum(hw_s + e, 0.0)
        agg = lax.dot_general(oh_d, msg, dn, preferred_element_type=f32)
        out = agg + hw * invd
        if l < 4:
            out = jnp.maximum(out, 0.0)
        hm = out + hm
    gids = g_ref[:, 0:1]
    rows = []
    for g in range(NG):
        red = jnp.max(jnp.where(gids == g, hm, -3e38), axis=0, keepdims=True)
        rows.append(jnp.where(red < -1e30, 0.0, red))
    hg = jnp.concatenate(rows, axis=0)
    o_ref[...] = jnp.dot(hg, wp_ref[...], preferred_element_type=f32) + bp_ref[0:1, :]


def _tc_motif(hm, srcb, dstb, attrm, wg5, weg5, bg8, m2gb, wp, bp8):
    return pl.pallas_call(
        _motif_body,
        out_shape=jax.ShapeDtypeStruct((NG, 128), jnp.float32),
    )(hm, srcb, dstb, attrm, wg5, weg5, bg8, m2gb, wp, bp8)


def kernel(logits, atom_mask, atom_edge_index, atom_edge_attr, atom2motif,
           motif_edge_index, motif_edge_attr, motif2graph,
           Wm, Wem, bm, Wg, Weg, bg, Wp, bp):
    f32 = jnp.float32
    i32 = jnp.int32
    B, N, _ = logits.shape
    n_atoms = B * N

    h0 = jnp.where(atom_mask.reshape(-1, 1), logits.reshape(n_atoms, EMB), 0.0)
    h0 = jnp.pad(h0, ((0, NP - n_atoms), (0, D - EMB)))
    h3 = h0.reshape(NP, NCH, DC).transpose(1, 0, 2)

    ne0 = atom_edge_index.shape[1]
    src = jnp.pad(atom_edge_index[0].astype(i32), (0, NE - ne0)).reshape(NBE, 128)
    dst = jnp.pad(atom_edge_index[1].astype(i32), (0, NE - ne0),
                  constant_values=DUMP).reshape(NBE, 128)
    attr = jnp.pad(atom_edge_attr.astype(f32), ((0, NE - ne0), (0, 128 - 16)))

    wm_r = jnp.pad(Wm, ((0, 0), (0, D - EMB), (0, D - EMB)))
    wm_r = wm_r.reshape(5, NCH, DC, NCH, DC).transpose(0, 1, 3, 2, 4)
    bm_b = jnp.broadcast_to(
        jnp.pad(bm, ((0, 0), (0, D - EMB))).reshape(5, NCH, 1, DC),
        (5, NCH, 8, DC))
    wem_r = jnp.pad(Wem, ((0, 0), (0, 128 - 16), (0, D - EMB)))
    wem_r = wem_r.reshape(5, 128, NCH, DC).transpose(0, 2, 1, 3)
    wem_r = wem_r.reshape(5 * NCH, 128, DC)

    a2m = jnp.pad(atom2motif.astype(i32), (0, NP - n_atoms),
                  constant_values=NMP - 1)
    a2mb = jnp.broadcast_to(a2m[:, None], (NP, 128))

    nem0 = motif_edge_index.shape[1]
    src_m = jnp.pad(motif_edge_index[0].astype(i32), (0, NEM - nem0),
                    constant_values=500)
    dst_m = jnp.pad(motif_edge_index[1].astype(i32), (0, NEM - nem0),
                    constant_values=501)
    srcb = jnp.broadcast_to(src_m[:, None], (NEM, 128))
    dstb = jnp.broadcast_to(dst_m[:, None], (NEM, 128))
    attrm = jnp.pad(motif_edge_attr.astype(f32), ((0, NEM - nem0), (0, 128 - 16)))
    wg5 = jnp.pad(Wg, ((0, 0), (0, D - EMB), (0, D - EMB)))
    weg5 = jnp.pad(Weg, ((0, 0), (0, 128 - 16), (0, D - EMB)))
    bg8 = jnp.pad(bg, ((0, 3), (0, D - EMB)))
    m2g = jnp.pad(motif2graph.astype(i32), (0, NMP - 500),
                  constant_values=NG - 1)
    m2gb = jnp.broadcast_to(m2g[:, None], (NMP, 128))
    wp = jnp.pad(Wp, ((0, D - EMB), (0, 128 - Wp.shape[1])))
    bp8 = jnp.broadcast_to(jnp.pad(bp, (0, 128 - bp.shape[0]))[None, :], (8, 128))

    norm, invd = _sc_deg_norm(src, dst)
    e_all = _tc_matmul_e(attr, wem_r)
    e_flat = e_all.reshape(5 * NCH * NE, DC)
    h = h3.reshape(NCH * NP, DC)
    for l in range(5):
        hw3 = _tc_matmul_h(h.reshape(NCH, NP, DC), wm_r[l], bm_b[l])
        hw = hw3.reshape(NCH * NP, DC)
        h = _sc_edge[l](hw, e_flat, src, dst, norm, invd, h)

    hm3 = _tc_segmax_atoms(h.reshape(NCH, NP, DC), a2mb)
    hm = hm3.transpose(1, 0, 2).reshape(NMP, D)
    out = _tc_motif(hm, srcb, dstb, attrm, wg5, weg5, bg8, m2gb, wp, bp8)
    return out[:50, :12]


# speedup vs baseline: 1.3250x; 1.0019x over previous
import functools

import jax
import jax.numpy as jnp
from jax import lax
from jax.experimental import pallas as pl
from jax.experimental.pallas import tpu as pltpu
from jax.experimental.pallas import tpu_sc as plsc

EMB = 300
D = 320
DC = 80
NCH = 4
NP = 10240
NE = 163840
NBE = NE // 128
NBT = NBE // 16
NPAIR = NBT // 2
RPT = NP // 16
NBR = NP // 128
NMP = 512
NEM = 2048
NG = 64
DUMP = 10200

_mesh = plsc.VectorSubcoreMesh(core_axis_name="c", subcore_axis_name="s",
                               num_cores=2, num_subcores=16)
_sc_params = pltpu.CompilerParams(needs_layout_passes=False,
                                  use_tc_tiling_on_sc=False)


def _fill(ref, value):
    for k in range(8):
        ref[pl.ds(k * 16, 16)] = jnp.full((16,), value, jnp.float32)


def _rsqrt16(x):
    i = plsc.bitcast(x, jnp.int32)
    i = jnp.int32(0x5F3759DF) - (i >> 1)
    y = plsc.bitcast(i, jnp.float32)
    for _ in range(3):
        y = y * (1.5 - 0.5 * x * y * y)
    return y


@functools.partial(
    pl.kernel,
    out_type=(jax.ShapeDtypeStruct((NBE, 128), jnp.float32),
              jax.ShapeDtypeStruct((NBR, 128), jnp.float32)),
    mesh=_mesh,
    scratch_types=[
        pltpu.VMEM((128,), jnp.int32),
        pltpu.VMEM((128,), jnp.int32),
        pltpu.VMEM((128,), jnp.float32),
        pltpu.VMEM((128,), jnp.float32),
        pltpu.VMEM((NP,), jnp.float32),
        pltpu.VMEM_SHARED((NP,), jnp.float32),
        pltpu.SemaphoreType.DMA,
    ],
    compiler_params=_sc_params,
)
def _sc_deg_norm(src_hbm, dst_hbm, norm_hbm, invd_hbm,
                 si_v, di_v, ones_v, st_v, degl_v, deg_sh, sem):
    c = lax.axis_index("c")
    s = lax.axis_index("s")

    _fill(ones_v, 1.0)
    _fill(st_v, 0.0)
    for i in range(RPT // 128):
        pltpu.sync_copy(st_v, deg_sh.at[pl.ds(s * RPT + i * 128, 128)])
    plsc.subcore_barrier()

    def blk(b, carry):
        pltpu.sync_copy(dst_hbm.at[s * NBT + b], di_v)
        pltpu.sync_copy(ones_v, deg_sh.at[di_v], add=True)
        return carry

    lax.fori_loop(0, NBT, blk, 0)
    plsc.subcore_barrier()
    pltpu.sync_copy(deg_sh, degl_v)

    @pl.when(c == 0)
    def _():
        for i in range(RPT // 128):
            base = s * RPT + i * 128
            for k in range(8):
                dv = degl_v[pl.ds(base + k * 16, 16)] + 1.0
                st_v[pl.ds(k * 16, 16)] = 1.0 / dv
            pltpu.sync_copy(st_v, invd_hbm.at[s * (RPT // 128) + i])

    def nblk(b, carry):
        g = (c * 16 + s) * (NBT // 2) + b
        pltpu.sync_copy(src_hbm.at[g], si_v)
        pltpu.sync_copy(dst_hbm.at[g], di_v)
        for k in range(8):
            sidx = si_v[pl.ds(k * 16, 16)]
            didx = di_v[pl.ds(k * 16, 16)]
            dsv = plsc.load_gather(degl_v, [sidx]) + 1.0
            ddv = plsc.load_gather(degl_v, [didx]) + 1.0
            st_v[pl.ds(k * 16, 16)] = _rsqrt16(dsv * ddv)
        pltpu.sync_copy(st_v, norm_hbm.at[g])
        return carry

    lax.fori_loop(0, NBT // 2, nblk, 0)


def _make_sc_edge(relu, layer):
    @functools.partial(
        pl.kernel,
        out_type=jax.ShapeDtypeStruct((NCH * NP, DC), jnp.float32),
        mesh=_mesh,
        scratch_types=[
            pltpu.VMEM((128,), jnp.int32),
            pltpu.VMEM((128,), jnp.int32),
            pltpu.VMEM((8, 128), jnp.int32),
            pltpu.VMEM((128,), jnp.float32),
            pltpu.VMEM((128,), jnp.float32),
            pltpu.VMEM((128,), jnp.float32),
            pltpu.VMEM((128, DC), jnp.float32),
            pltpu.VMEM((128, DC), jnp.float32),
            pltpu.VMEM((128, DC), jnp.float32),
            pltpu.VMEM((128, DC), jnp.float32),
            pltpu.VMEM((128, DC), jnp.float32),
            pltpu.VMEM((128, DC), jnp.float32),
            pltpu.VMEM_SHARED((NP, DC), jnp.float32),
            pltpu.SemaphoreType.DMA,
            pltpu.SemaphoreType.DMA,
            pltpu.SemaphoreType.DMA,
            pltpu.SemaphoreType.DMA,
            pltpu.SemaphoreType.DMA,
            pltpu.SemaphoreType.DMA,
        ],
        name=f"sc_edge_l{layer}",
        compiler_params=_sc_params,
    )
    def k(hw_hbm, e_hbm, src_hbm, dst_hbm, norm_hbm, invd_hbm, h_hbm, out_hbm,
          siA, siB, di_v, nvA, nvB, iv_v, rA, rB, eA, eB, mA, mB, agg_sh,
          isA, isB, gsA, gsB, ssA, ssB):
        c = lax.axis_index("c")
        s = lax.axis_index("s")


        def compute_block(rows_v, ev_v, nv_v, msg_v):
            def row(j, carry2):
                jj = jnp.full((16,), 0, jnp.int32) + j
                nrm = plsc.load_gather(nv_v, [jj])
                for k2 in range(DC // 16):
                    v = rows_v[j, pl.ds(k2 * 16, 16)] + ev_v[j, pl.ds(k2 * 16, 16)]
                    msg_v[j, pl.ds(k2 * 16, 16)] = jnp.maximum(v, 0.0) * nrm
                return carry2
            lax.fori_loop(0, 128, row, 0)

        for p in range(2):
            ch = 2 * p + c
            rowbase = ch * NP
            ebase = (NCH * layer + ch) * NE

            def zrow(j, carry):
                for k2 in range(DC // 16):
                    mA[j, pl.ds(k2 * 16, 16)] = jnp.zeros((16,), jnp.float32)
                return carry

            lax.fori_loop(0, 128, zrow, 0)
            for i in range(RPT // 128):
                pltpu.sync_copy(mA, agg_sh.at[pl.ds(s * RPT + i * 128, 128)])
            plsc.subcore_barrier()

            def step(b, si_p, si_q, nv_p, nv_q, r_p, r_q, e_p, e_q, m_p,
                     is_p, is_q, gs_p, gs_q, ss_p, skip_sw):
                pltpu.make_async_copy(hw_hbm.at[si_p], r_p, gs_p).wait()
                pltpu.make_async_copy(e_hbm.at[pl.ds(ebase, 128)], e_p, gs_p).wait()
                pltpu.make_async_copy(norm_hbm.at[0], nv_p, gs_p).wait()
                @pl.when(b + 2 < NBT)
                def _():
                    pltpu.async_copy(src_hbm.at[s * NBT + b + 2], si_p, is_p)
                    pltpu.async_copy(dst_hbm.at[s * NBT + b + 2],
                                     di_v.at[lax.rem(b + 2, 8)], is_p)
                @pl.when(b + 1 < NBT)
                def _():
                    pltpu.make_async_copy(src_hbm.at[0], si_q, is_q).wait()
                    pltpu.make_async_copy(dst_hbm.at[0],
                                          di_v.at[0], is_q).wait()
                    def srow(k2, carry):
                        si_q[pl.ds(k2 * 16, 16)] = (
                            si_q[pl.ds(k2 * 16, 16)] + rowbase)
                        return carry
                    lax.fori_loop(0, 8, srow, 0)
                    off1 = s * NBT * 128 + (b + 1) * 128
                    pltpu.async_copy(hw_hbm.at[si_q], r_q, gs_q)
                    pltpu.async_copy(e_hbm.at[pl.ds(ebase + off1, 128)], e_q, gs_q)
                    pltpu.async_copy(norm_hbm.at[s * NBT + b + 1], nv_q, gs_q)
                @pl.when(jnp.logical_not(skip_sw))
                def _():
                    pltpu.make_async_copy(m_p, agg_sh.at[di_v.at[0]], ss_p).wait()
                compute_block(r_p, e_p, nv_p, m_p)
                pltpu.async_copy(m_p, agg_sh.at[di_v.at[lax.rem(b, 8)]],
                                 ss_p, add=True)

            pltpu.async_copy(src_hbm.at[s * NBT], siA, isA)
            pltpu.async_copy(dst_hbm.at[s * NBT], di_v.at[0], isA)
            pltpu.async_copy(src_hbm.at[s * NBT + 1], siB, isB)
            pltpu.async_copy(dst_hbm.at[s * NBT + 1], di_v.at[1], isB)
            pltpu.make_async_copy(src_hbm.at[0], siA, isA).wait()
            pltpu.make_async_copy(dst_hbm.at[0], di_v.at[0], isA).wait()

            def srow0(k2, carry):
                siA[pl.ds(k2 * 16, 16)] = siA[pl.ds(k2 * 16, 16)] + rowbase
                return carry
            lax.fori_loop(0, 8, srow0, 0)
            pltpu.async_copy(hw_hbm.at[siA], rA, gsA)
            pltpu.async_copy(e_hbm.at[pl.ds(ebase + s * NBT * 128, 128)], eA, gsA)
            pltpu.async_copy(norm_hbm.at[s * NBT], nvA, gsA)

            def pair(i, carry):
                step(2 * i, siA, siB, nvA, nvB, rA, rB, eA, eB, mA,
                     isA, isB, gsA, gsB, ssA, i == 0)
                step(2 * i + 1, siB, siA, nvB, nvA, rB, rA, eB, eA, mB,
                     isB, isA, gsB, gsA, ssB, i == 0)
                return carry

            lax.fori_loop(0, NPAIR, pair, 0)
            pltpu.make_async_copy(mA, agg_sh.at[di_v.at[0]], ssA).wait()
            pltpu.make_async_copy(mB, agg_sh.at[di_v.at[0]], ssB).wait()
            plsc.subcore_barrier()

            for i in range(RPT // 128):
                r0 = s * RPT + i * 128
                pltpu.sync_copy(agg_sh.at[pl.ds(r0, 128)], rA)
                pltpu.sync_copy(hw_hbm.at[pl.ds(rowbase + r0, 128)], eA)
                pltpu.sync_copy(h_hbm.at[pl.ds(rowbase + r0, 128)], mA)
                pltpu.sync_copy(invd_hbm.at[s * (RPT // 128) + i], iv_v)

                def crow(j, carry):
                    jj = jnp.full((16,), 0, jnp.int32) + j
                    idv = plsc.load_gather(iv_v, [jj])
                    for k2 in range(DC // 16):
                        o = rA[j, pl.ds(k2 * 16, 16)] + eA[j, pl.ds(k2 * 16, 16)] * idv
                        if relu:
                            o = jnp.maximum(o, 0.0)
                        rA[j, pl.ds(k2 * 16, 16)] = o + mA[j, pl.ds(k2 * 16, 16)]
                    return carry

                lax.fori_loop(0, 128, crow, 0)
                pltpu.sync_copy(rA, out_hbm.at[pl.ds(rowbase + r0, 128)])

    return k


_sc_edge = [_make_sc_edge(l < 4, l) for l in range(5)]


def _mm_h_body(a0, a1, a2, a3, w_ref, b_ref, o_ref):
    w = w_ref[...]
    acc = jnp.dot(a0[...], w[0, 0], preferred_element_type=jnp.float32)
    for k, a in enumerate((a1, a2, a3)):
        acc = acc + jnp.dot(a[...], w[k + 1, 0],
                            preferred_element_type=jnp.float32)
    o_ref[...] = acc + b_ref[0, 0:1, :]


def _tc_matmul_h(h, wr, bb):
    bm = 1024
    nb = NP // bm

    def mk(k):
        return pl.BlockSpec((bm, DC), lambda i, c, k=k: (k * nb + i, 0))

    return pl.pallas_call(
        _mm_h_body,
        grid=(nb, NCH),
        in_specs=[
            mk(0), mk(1), mk(2), mk(3),
            pl.BlockSpec((NCH, 1, DC, DC), lambda i, c: (0, c, 0, 0)),
            pl.BlockSpec((1, 8, DC), lambda i, c: (c, 0, 0)),
        ],
        out_specs=pl.BlockSpec((bm, DC), lambda i, c: (c * nb + i, 0)),
        out_shape=jax.ShapeDtypeStruct((NCH * NP, DC), jnp.float32),
    )(h, h, h, h, wr, bb)


def _mm_e_body(a_ref, w_ref, o_ref):
    o_ref[...] = jnp.dot(a_ref[...], w_ref[0],
                         preferred_element_type=jnp.float32)


def _tc_matmul_e(attr, wr):
    bm = 2048
    nb = NE // bm
    ng = 5 * NCH
    return pl.pallas_call(
        _mm_e_body,
        grid=(nb, ng),
        in_specs=[
            pl.BlockSpec((bm, 128), lambda i, g: (i, 0)),
            pl.BlockSpec((1, 128, DC), lambda i, g: (g, 0, 0)),
        ],
        out_specs=pl.BlockSpec((bm, DC), lambda i, g: (g * nb + i, 0)),
        out_shape=jax.ShapeDtypeStruct((ng * NE, DC), jnp.float32),
    )(attr, wr)


def _segmax_atoms_body(x_ref, m_ref, o_ref):
    mb = pl.program_id(0)
    x = x_ref[...]
    ids = m_ref[:, 0:1]
    for jj in range(8):
        mask = ids == (mb * 8 + jj)
        red = jnp.max(jnp.where(mask, x, -3e38), axis=0, keepdims=True)
        red = jnp.where(red < -1e30, 0.0, red)
        o_ref[0, jj:jj + 1, :] = red


def _tc_segmax_atoms(h, a2mb):
    return pl.pallas_call(
        _segmax_atoms_body,
        grid=(NMP // 8, NCH),
        in_specs=[
            pl.BlockSpec((NP, DC), lambda mb, c: (c, 0)),
            pl.BlockSpec((NP, 128), lambda mb, c: (0, 0)),
        ],
        out_specs=pl.BlockSpec((1, 8, DC), lambda mb, c: (c, mb, 0)),
        out_shape=jax.ShapeDtypeStruct((NCH, NMP, DC), jnp.float32),
    )(h, a2mb)


def _motif_body(hm_ref, s_ref, d_ref, a_ref, wg_ref, weg_ref, bg_ref, g_ref,
                wp_ref, bp_ref, o_ref):
    f32 = jnp.float32
    iota = lax.broadcasted_iota(jnp.int32, (NEM, NMP), 1)
    oh_s = (jnp.broadcast_to(s_ref[:, 0:1], (NEM, NMP)) == iota).astype(f32)
    oh_d = (jnp.broadcast_to(d_ref[:, 0:1], (NEM, NMP)) == iota).astype(f32)
    ones = jnp.ones((NEM, 128), f32)
    dn = (((0,), (0,)), ((), ()))
    degc = lax.dot_general(oh_d, ones, dn, preferred_element_type=f32) + 1.0
    deg_s = jnp.dot(oh_s, degc, preferred_element_type=f32)[:, 0:1]
    deg_d = jnp.dot(oh_d, degc, preferred_element_type=f32)[:, 0:1]
    nrm = lax.rsqrt(deg_s * deg_d)
    invd = 1.0 / degc[:, 0:1]
    hm = hm_ref[...]
    am = a_ref[...]
    for l in range(5):
        hw = jnp.dot(hm, wg_ref[l], preferred_element_type=f32) + bg_ref[l:l + 1, :]
        e = jnp.dot(am, weg_ref[l], preferred_element_type=f32)
        hw_s = jnp.dot(oh_s, hw, preferred_element_type=f32)
        msg = nrm * jnp.maximum(hw_s + e, 0.0)
        agg = lax.dot_general(oh_d, msg, dn, preferred_element_type=f32)
        out = agg + hw * invd
        if l < 4:
            out = jnp.maximum(out, 0.0)
        hm = out + hm
    gids = g_ref[:, 0:1]
    rows = []
    for g in range(NG):
        red = jnp.max(jnp.where(gids == g, hm, -3e38), axis=0, keepdims=True)
        rows.append(jnp.where(red < -1e30, 0.0, red))
    hg = jnp.concatenate(rows, axis=0)
    o_ref[...] = jnp.dot(hg, wp_ref[...], preferred_element_type=f32) + bp_ref[0:1, :]


def _tc_motif(hm, srcb, dstb, attrm, wg5, weg5, bg8, m2gb, wp, bp8):
    return pl.pallas_call(
        _motif_body,
        out_shape=jax.ShapeDtypeStruct((NG, 128), jnp.float32),
    )(hm, srcb, dstb, attrm, wg5, weg5, bg8, m2gb, wp, bp8)


def kernel(logits, atom_mask, atom_edge_index, atom_edge_attr, atom2motif,
           motif_edge_index, motif_edge_attr, motif2graph,
           Wm, Wem, bm, Wg, Weg, bg, Wp, bp):
    f32 = jnp.float32
    i32 = jnp.int32
    B, N, _ = logits.shape
    n_atoms = B * N

    h0 = jnp.where(atom_mask.reshape(-1, 1), logits.reshape(n_atoms, EMB), 0.0)
    h0 = jnp.pad(h0, ((0, NP - n_atoms), (0, D - EMB)))
    hflat = h0.reshape(NP, NCH, DC).transpose(1, 0, 2).reshape(NCH * NP, DC)

    ne0 = atom_edge_index.shape[1]
    src = jnp.pad(atom_edge_index[0].astype(i32), (0, NE - ne0)).reshape(NBE, 128)
    dst = jnp.pad(atom_edge_index[1].astype(i32), (0, NE - ne0),
                  constant_values=DUMP).reshape(NBE, 128)
    attr = jnp.pad(atom_edge_attr.astype(f32), ((0, NE - ne0), (0, 128 - 16)))

    wm_r = jnp.pad(Wm, ((0, 0), (0, D - EMB), (0, D - EMB)))
    wm_r = wm_r.reshape(5, NCH, DC, NCH, DC).transpose(0, 1, 3, 2, 4)
    bm_b = jnp.broadcast_to(
        jnp.pad(bm, ((0, 0), (0, D - EMB))).reshape(5, NCH, 1, DC),
        (5, NCH, 8, DC))
    wem_r = jnp.pad(Wem, ((0, 0), (0, 128 - 16), (0, D - EMB)))
    wem_r = wem_r.reshape(5, 128, NCH, DC).transpose(0, 2, 1, 3)
    wem_r = wem_r.reshape(5 * NCH, 128, DC)

    a2m = jnp.pad(atom2motif.astype(i32), (0, NP - n_atoms),
                  constant_values=NMP - 1)
    a2mb = jnp.broadcast_to(a2m[:, None], (NP, 128))

    nem0 = motif_edge_index.shape[1]
    src_m = jnp.pad(motif_edge_index[0].astype(i32), (0, NEM - nem0),
                    constant_values=500)
    dst_m = jnp.pad(motif_edge_index[1].astype(i32), (0, NEM - nem0),
                    constant_values=501)
    srcb = jnp.broadcast_to(src_m[:, None], (NEM, 128))
    dstb = jnp.broadcast_to(dst_m[:, None], (NEM, 128))
    attrm = jnp.pad(motif_edge_attr.astype(f32), ((0, NEM - nem0), (0, 128 - 16)))
    wg5 = jnp.pad(Wg, ((0, 0), (0, D - EMB), (0, D - EMB)))
    weg5 = jnp.pad(Weg, ((0, 0), (0, 128 - 16), (0, D - EMB)))
    bg8 = jnp.pad(bg, ((0, 3), (0, D - EMB)))
    m2g = jnp.pad(motif2graph.astype(i32), (0, NMP - 500),
                  constant_values=NG - 1)
    m2gb = jnp.broadcast_to(m2g[:, None], (NMP, 128))
    wp = jnp.pad(Wp, ((0, D - EMB), (0, 128 - Wp.shape[1])))
    bp8 = jnp.broadcast_to(jnp.pad(bp, (0, 128 - bp.shape[0]))[None, :], (8, 128))

    norm, invd = _sc_deg_norm(src, dst)
    e_flat = _tc_matmul_e(attr, wem_r)
    h = hflat
    for l in range(5):
        hw = _tc_matmul_h(h, wm_r[l], bm_b[l])
        h = _sc_edge[l](hw, e_flat, src, dst, norm, invd, h)

    hm3 = _tc_segmax_atoms(h, a2mb)
    hm = hm3.transpose(1, 0, 2).reshape(NMP, D)
    out = _tc_motif(hm, srcb, dstb, attrm, wg5, weg5, bg8, m2gb, wp, bp8)
    return out[:50, :12]
